# Initial kernel scaffold; baseline (speedup 1.0000x reference)
import jax
import jax.numpy as jnp
from jax.experimental import pallas as pl


def kernel(X, Z, combo_w, nl_w):
    raise NotImplementedError("write your pallas kernel here")


# speedup vs baseline: 55.5518x; 55.5518x over previous
import functools

import jax
import jax.numpy as jnp
import numpy as np
from jax import lax
from jax.experimental import pallas as pl
from jax.experimental.pallas import tpu as pltpu
from jax.experimental.pallas import tpu_sc as plsc

N_ATOMS = 50000
M_NBRS = 16
NBR_CUTOFF = 8.0
N_CELLS = 125

CHUNK = 2048
N_CHUNKS = 25
N_SCAN = N_CHUNKS * CHUNK

BA = 512
N_BLOCKS = 98
N_EPAD = N_BLOCKS * BA

_BIGF = np.float32(1e30)
_BIGI = np.int32(2**30)
_BIG8 = np.float32(1e9)
_BIG9 = np.float32(2e9)
_FAR = np.float32(1e5)

_r = np.arange(0.0, 40.0, 8.0, dtype=np.float32)
_mesh = np.meshgrid(_r, _r, _r)
_CELLS = np.transpose(np.stack(_mesh)).reshape(N_CELLS, 3).astype(np.float32)
_d_cc = ((_CELLS[:, None, :] - _CELLS[None, :, :]) ** 2).sum(-1)
_NBR_CELLS = np.argsort(_d_cc, axis=1, kind="stable")[:, :8].astype(np.int32)

_CELLS_LANE = np.zeros((8, 128), np.float32)
_CELLS_LANE[0:3, :N_CELLS] = _CELLS.T
_CELLS_LANE[0:3, N_CELLS:] = _FAR
_CELLS_SUB = np.zeros((128, 8), np.float32)
_CELLS_SUB[:N_CELLS, 0:3] = _CELLS
_CELLS_SUB[N_CELLS:, 0:3] = _FAR


def _topk_cells_kernel(xt_ref, csub_ref, out_ref, rv_ref, ri_ref):
    pid = pl.program_id(0)

    @pl.when(pid == 0)
    def _init():
        rv_ref[...] = jnp.full((128, 16), _BIGF, jnp.float32)
        ri_ref[...] = jnp.full((128, 16), _BIGI, jnp.int32)

    xs = xt_ref[0:1, :]
    ys = xt_ref[1:2, :]
    zs = xt_ref[2:3, :]
    cx = csub_ref[:, 0:1]
    cy = csub_ref[:, 1:2]
    cz = csub_ref[:, 2:3]
    d = (cx - xs) ** 2 + (cy - ys) ** 2 + (cz - zs) ** 2
    gidx = pid * CHUNK + lax.broadcasted_iota(jnp.int32, (128, CHUNK), 1)

    vals, idxs = [], []
    for _ in range(M_NBRS):
        m = jnp.min(d, axis=1, keepdims=True)
        mi = jnp.min(jnp.where(d == m, gidx, _BIGI), axis=1, keepdims=True)
        vals.append(m)
        idxs.append(mi)
        d = jnp.where(gidx == mi, _BIGF, d)
    av = jnp.concatenate([rv_ref[...]] + vals, axis=1)
    ai = jnp.concatenate([ri_ref[...]] + idxs, axis=1)

    nvals, nidxs = [], []
    for _ in range(M_NBRS):
        m = jnp.min(av, axis=1, keepdims=True)
        mi = jnp.min(jnp.where(av == m, ai, _BIGI), axis=1, keepdims=True)
        nvals.append(m)
        nidxs.append(mi)
        av = jnp.where((av == m) & (ai == mi), _BIGF, av)
    rv_ref[...] = jnp.concatenate(nvals, axis=1)
    ri_ref[...] = jnp.concatenate(nidxs, axis=1)
    out_ref[...] = ri_ref[...]


def _run_topk_cells(xt_pad):
    return pl.pallas_call(
        _topk_cells_kernel,
        grid=(N_CHUNKS,),
        in_specs=[
            pl.BlockSpec((8, CHUNK), lambda i: (0, i)),
            pl.BlockSpec((128, 8), lambda i: (0, 0)),
        ],
        out_specs=pl.BlockSpec((128, M_NBRS), lambda i: (0, 0)),
        out_shape=jax.ShapeDtypeStruct((128, M_NBRS), jnp.int32),
        scratch_shapes=[
            pltpu.VMEM((128, M_NBRS), jnp.float32),
            pltpu.VMEM((128, M_NBRS), jnp.int32),
        ],
    )(xt_pad, jnp.asarray(_CELLS_SUB))


def _dedup_kernel(ids_ref, idsf_ref):
    ids = ids_ref[...]
    lane = lax.broadcasted_iota(jnp.int32, (128, 128), 1)
    dup = jnp.zeros((128, 128), jnp.bool_)
    for j in range(1, 128):
        idj = ids[:, j:j + 1]
        dupj = jnp.any((ids == idj) & (lane < j), axis=1, keepdims=True)
        dup = dup | (dupj & (lane == j))
    clean = jnp.where(dup, -1, ids)
    idsf_ref[...] = clean.astype(jnp.float32)


def _run_dedup(ain):
    return pl.pallas_call(
        _dedup_kernel,
        out_shape=jax.ShapeDtypeStruct((128, 128), jnp.float32),
    )(ain)


_NGATHER = 128 * 128
_B_PER_W = _NGATHER // 32


def _sc_gather(table128, gidx_flat):
    mesh = plsc.VectorSubcoreMesh(
        core_axis_name="c", subcore_axis_name="s", num_cores=2, num_subcores=16
    )

    @functools.partial(
        pl.kernel,
        out_type=jax.ShapeDtypeStruct((_NGATHER, 128), jnp.float32),
        mesh=mesh,
        scratch_types=[
            pltpu.VMEM((_B_PER_W,), jnp.int32),
            pltpu.VMEM((_B_PER_W, 128), jnp.float32),
            pltpu.SemaphoreType.DMA,
        ],
    )
    def gat(table_hbm, idx_hbm, out_hbm, idx_v, rows_v, sem):
        wid = lax.axis_index("s") * 2 + lax.axis_index("c")
        base = wid * _B_PER_W
        pltpu.sync_copy(idx_hbm.at[pl.ds(base, _B_PER_W)], idx_v)
        pltpu.async_copy(table_hbm.at[idx_v], rows_v, sem).wait()
        pltpu.sync_copy(rows_v, out_hbm.at[pl.ds(base, _B_PER_W)])

    return gat(table128, gidx_flat)


def _energy_terms(d, w0, w1, w2, w3, w4):
    rep = jnp.where(d < 0.0, d * d, jnp.zeros_like(d))
    hyd = jnp.where(d < 0.5, jnp.ones_like(d),
                    jnp.where(d < 1.5, 1.5 - d, jnp.zeros_like(d)))
    hb = jnp.where(d < -0.7, jnp.ones_like(d),
                   jnp.where(d < 0.0, (1.0 / 0.7) * (0.0 - d), jnp.zeros_like(d)))
    g1 = jnp.exp(-((d / 0.5) ** 2))
    g2 = jnp.exp(-(((d - 3.0) / 2.0) ** 2))
    inter = w0 * rep + w1 * hyd + w2 * hb + w3 * g1 + w4 * g2
    return jnp.where(d < 8.0, inter, jnp.zeros_like(d))


def _energy_kernel(xc_ref, clane_ref, idsf_ref, gx_ref, gy_ref, gz_ref,
                   par_ref, out_ref, e_ref, acc_ref):
    pid = pl.program_id(0)

    @pl.when(pid == 0)
    def _init():
        acc_ref[0, 0] = jnp.float32(0.0)

    xa = xc_ref[:, 0:1]
    ya = xc_ref[:, 1:2]
    za = xc_ref[:, 2:3]
    cxr = clane_ref[0:1, :]
    cyr = clane_ref[1:2, :]
    czr = clane_ref[2:3, :]
    d2 = (xa - cxr) ** 2 + (ya - cyr) ** 2 + (za - czr) ** 2
    lane = lax.broadcasted_iota(jnp.int32, (BA, 128), 1)
    m = jnp.min(d2, axis=1, keepdims=True)
    cidx = jnp.min(jnp.where(d2 == m, lane, 999), axis=1, keepdims=True)
    oh = (lane == cidx).astype(jnp.float32)

    hi = jax.lax.Precision.HIGHEST
    ids = jnp.dot(oh, idsf_ref[...], precision=hi)
    cx = jnp.dot(oh, gx_ref[...], precision=hi)
    cy = jnp.dot(oh, gy_ref[...], precision=hi)
    cz = jnp.dot(oh, gz_ref[...], precision=hi)
    d = (xa - cx) ** 2 + (ya - cy) ** 2 + (za - cz) ** 2

    sub = lax.broadcasted_iota(jnp.int32, (BA, 1), 0)
    aid = pid * BA + sub
    valid = (ids >= 0.0) & (ids != aid.astype(jnp.float32))
    dm = jnp.where(valid, d, _BIG8)

    w0 = par_ref[0, 0]
    w1 = par_ref[0, 1]
    w2 = par_ref[0, 2]
    w3 = par_ref[0, 3]
    w4 = par_ref[0, 4]
    s_nl = 1.0 + par_ref[0, 5]

    th = _energy_terms(dm, w0, w1, w2, w3, w4)
    s_direct = jnp.sum(th, axis=1, keepdims=True)
    c8 = jnp.sum((dm < 8.0).astype(jnp.float32), axis=1, keepdims=True)
    e_ref[...] = s_direct

    @pl.when(jnp.max(c8) > 16.0)
    def _rank_fix():
        dcur = dm
        s = jnp.zeros((BA, 1), jnp.float32)
        taken = jnp.zeros((BA, 1), jnp.float32)
        for _ in range(M_NBRS):
            mv = jnp.min(dcur, axis=1, keepdims=True)
            cnt = jnp.sum((dcur == mv).astype(jnp.float32), axis=1, keepdims=True)
            take = jnp.clip(16.0 - taken, 0.0, cnt)
            s = s + take * _energy_terms(mv, w0, w1, w2, w3, w4)
            taken = taken + take
            dcur = jnp.where(dcur == mv, _BIG9, dcur)
        e_ref[...] = jnp.where(c8 > 16.0, s, s_direct)

    ok = (aid < N_ATOMS).astype(jnp.float32)
    blocksum = jnp.sum(e_ref[...] * ok)
    acc_ref[0, 0] = acc_ref[0, 0] + blocksum / s_nl
    out_ref[0, 0] = acc_ref[0, 0]


def _run_energy(xc_pad, idsf, gx, gy, gz, params):
    return pl.pallas_call(
        _energy_kernel,
        grid=(N_BLOCKS,),
        in_specs=[
            pl.BlockSpec((BA, 4), lambda i: (i, 0)),
            pl.BlockSpec((8, 128), lambda i: (0, 0)),
            pl.BlockSpec((128, 128), lambda i: (0, 0)),
            pl.BlockSpec((128, 128), lambda i: (0, 0)),
            pl.BlockSpec((128, 128), lambda i: (0, 0)),
            pl.BlockSpec((128, 128), lambda i: (0, 0)),
            pl.BlockSpec(memory_space=pltpu.SMEM),
        ],
        out_specs=pl.BlockSpec(memory_space=pltpu.SMEM),
        out_shape=jax.ShapeDtypeStruct((1, 1), jnp.float32),
        scratch_shapes=[
            pltpu.VMEM((BA, 1), jnp.float32),
            pltpu.SMEM((1, 1), jnp.float32),
        ],
    )(xc_pad, jnp.asarray(_CELLS_LANE), idsf, gx, gy, gz, params)


def kernel(X, Z, combo_w, nl_w):
    X = X.astype(jnp.float32)

    xt = jnp.concatenate(
        [X.T, jnp.full((3, N_SCAN - N_ATOMS), _FAR, jnp.float32)], axis=1)
    xt_pad = jnp.concatenate([xt, jnp.zeros((5, N_SCAN), jnp.float32)], axis=0)
    xc = jnp.concatenate([X, jnp.zeros((N_ATOMS, 1), jnp.float32)], axis=1)
    xc_pad = jnp.concatenate(
        [xc, jnp.zeros((N_EPAD - N_ATOMS, 4), jnp.float32)], axis=0)
    params = jnp.zeros((1, 8), jnp.float32)
    params = params.at[0, 0:5].set(combo_w).at[0, 5].set(nl_w[0])

    closest = _run_topk_cells(xt_pad)

    ain = closest[:N_CELLS][jnp.asarray(_NBR_CELLS)].reshape(N_CELLS, 128)
    ain = jnp.concatenate(
        [ain, jnp.full((3, 128), -1, jnp.int32)], axis=0)

    idsf = _run_dedup(ain)

    table128 = jnp.concatenate(
        [X, jnp.zeros((N_ATOMS, 125), jnp.float32)], axis=1)
    rows = _sc_gather(table128, jnp.maximum(ain, 0).reshape(_NGATHER))
    gx = rows[:, 0].reshape(128, 128)
    gy = rows[:, 1].reshape(128, 128)
    gz = rows[:, 2].reshape(128, 128)

    out = _run_energy(xc_pad, idsf, gx, gy, gz, params)
    return out[0, 0]


# speedup vs baseline: 66.0458x; 1.1889x over previous
import functools

import jax
import jax.numpy as jnp
import numpy as np
from jax import lax
from jax.experimental import pallas as pl
from jax.experimental.pallas import tpu as pltpu
from jax.experimental.pallas import tpu_sc as plsc

N_ATOMS = 50000
M_NBRS = 16
NBR_CUTOFF = 8.0
N_CELLS = 125

CHUNK = 2048
N_CHUNKS = 25
N_SCAN = N_CHUNKS * CHUNK

BA = 512
N_BLOCKS = 98
N_EPAD = N_BLOCKS * BA

_BIGF = np.float32(1e30)
_BIGI = np.int32(2**30)
_BIG8 = np.float32(1e9)
_BIG9 = np.float32(2e9)
_FAR = np.float32(1e5)

_r = np.arange(0.0, 40.0, 8.0, dtype=np.float32)
_mesh = np.meshgrid(_r, _r, _r)
_CELLS = np.transpose(np.stack(_mesh)).reshape(N_CELLS, 3).astype(np.float32)
_d_cc = ((_CELLS[:, None, :] - _CELLS[None, :, :]) ** 2).sum(-1)
_NBR_CELLS = np.argsort(_d_cc, axis=1, kind="stable")[:, :8].astype(np.int32)

_CELLS_LANE = np.zeros((8, 128), np.float32)
_CELLS_LANE[0:3, :N_CELLS] = _CELLS.T
_CELLS_LANE[0:3, N_CELLS:] = _FAR
_CELLS_SUB = np.zeros((128, 8), np.float32)
_CELLS_SUB[:N_CELLS, 0:3] = _CELLS
_CELLS_SUB[N_CELLS:, 0:3] = _FAR


def _topk_cells_kernel(xt_ref, csub_ref, out_ref, rv_ref, ri_ref):
    pid = pl.program_id(0)

    @pl.when(pid == 0)
    def _init():
        rv_ref[...] = jnp.full((128, 16), _BIGF, jnp.float32)
        ri_ref[...] = jnp.full((128, 16), _BIG8, jnp.float32)

    xs = xt_ref[0:1, :]
    ys = xt_ref[1:2, :]
    zs = xt_ref[2:3, :]
    cx = csub_ref[:, 0:1]
    cy = csub_ref[:, 1:2]
    cz = csub_ref[:, 2:3]
    d = (cx - xs) ** 2 + (cy - ys) ** 2 + (cz - zs) ** 2
    gidx = (pid * CHUNK).astype(jnp.float32) + lax.broadcasted_iota(
        jnp.int32, (128, CHUNK), 1).astype(jnp.float32)

    vals, idxs = [], []
    for _ in range(M_NBRS):
        m = jnp.min(d, axis=1, keepdims=True)
        mi = jnp.min(jnp.where(d == m, gidx, _BIG8), axis=1, keepdims=True)
        vals.append(m)
        idxs.append(mi)
        d = jnp.where(gidx == mi, _BIGF, d)
    av = jnp.concatenate([rv_ref[...]] + vals, axis=1)
    ai = jnp.concatenate([ri_ref[...]] + idxs, axis=1)

    nvals, nidxs = [], []
    for _ in range(M_NBRS):
        m = jnp.min(av, axis=1, keepdims=True)
        mi = jnp.min(jnp.where(av == m, ai, _BIG8), axis=1, keepdims=True)
        nvals.append(m)
        nidxs.append(mi)
        av = jnp.where((av == m) & (ai == mi), _BIGF, av)
    rv_ref[...] = jnp.concatenate(nvals, axis=1)
    ri_ref[...] = jnp.concatenate(nidxs, axis=1)
    out_ref[...] = ri_ref[...].astype(jnp.int32)


def _run_topk_cells(xt_pad):
    return pl.pallas_call(
        _topk_cells_kernel,
        grid=(N_CHUNKS,),
        in_specs=[
            pl.BlockSpec((8, CHUNK), lambda i: (0, i)),
            pl.BlockSpec((128, 8), lambda i: (0, 0)),
        ],
        out_specs=pl.BlockSpec((128, M_NBRS), lambda i: (0, 0)),
        out_shape=jax.ShapeDtypeStruct((128, M_NBRS), jnp.int32),
        scratch_shapes=[
            pltpu.VMEM((128, M_NBRS), jnp.float32),
            pltpu.VMEM((128, M_NBRS), jnp.float32),
        ],
    )(xt_pad, jnp.asarray(_CELLS_SUB))


def _dedup_kernel(ids_ref, idsf_ref):
    ids = ids_ref[...]
    lane = lax.broadcasted_iota(jnp.int32, (128, 128), 1)
    dup = jnp.zeros((128, 128), jnp.bool_)
    for j in range(1, 128):
        idj = ids[:, j:j + 1]
        dupj = jnp.any((ids == idj) & (lane < j), axis=1, keepdims=True)
        dup = dup | (dupj & (lane == j))
    clean = jnp.where(dup, -1, ids)
    idsf_ref[...] = clean.astype(jnp.float32)


def _run_dedup(ain):
    return pl.pallas_call(
        _dedup_kernel,
        out_shape=jax.ShapeDtypeStruct((128, 128), jnp.float32),
    )(ain)


_NGATHER = 128 * 128
_B_PER_W = _NGATHER // 32


def _sc_gather(table128, gidx_flat):
    mesh = plsc.VectorSubcoreMesh(
        core_axis_name="c", subcore_axis_name="s", num_cores=2, num_subcores=16
    )

    @functools.partial(
        pl.kernel,
        out_type=jax.ShapeDtypeStruct((_NGATHER, 128), jnp.float32),
        mesh=mesh,
        scratch_types=[
            pltpu.VMEM((_B_PER_W,), jnp.int32),
            pltpu.VMEM((_B_PER_W, 128), jnp.float32),
            pltpu.SemaphoreType.DMA,
        ],
    )
    def gat(table_hbm, idx_hbm, out_hbm, idx_v, rows_v, sem):
        wid = lax.axis_index("s") * 2 + lax.axis_index("c")
        base = wid * _B_PER_W
        pltpu.sync_copy(idx_hbm.at[pl.ds(base, _B_PER_W)], idx_v)
        pltpu.async_copy(table_hbm.at[idx_v], rows_v, sem).wait()
        pltpu.sync_copy(rows_v, out_hbm.at[pl.ds(base, _B_PER_W)])

    return gat(table128, gidx_flat)


def _energy_terms(d, w1, w3, w4):
    hyd = jnp.where(d < 0.5, jnp.ones_like(d),
                    jnp.where(d < 1.5, 1.5 - d, jnp.zeros_like(d)))
    g1 = jnp.exp(-((d / 0.5) ** 2))
    g2 = jnp.exp(-(((d - 3.0) / 2.0) ** 2))
    inter = w1 * hyd + w3 * g1 + w4 * g2
    return jnp.where(d < 8.0, inter, jnp.zeros_like(d))


def _energy_kernel(xc_ref, clane_ref, idsf_ref, gx_ref, gy_ref, gz_ref,
                   par_ref, out_ref, e_ref, acc_ref):
    pid = pl.program_id(0)

    @pl.when(pid == 0)
    def _init():
        acc_ref[0, 0] = jnp.float32(0.0)

    xa = xc_ref[:, 0:1]
    ya = xc_ref[:, 1:2]
    za = xc_ref[:, 2:3]
    cxr = clane_ref[0:1, :]
    cyr = clane_ref[1:2, :]
    czr = clane_ref[2:3, :]
    d2 = (xa - cxr) ** 2 + (ya - cyr) ** 2 + (za - czr) ** 2
    lane = lax.broadcasted_iota(jnp.int32, (BA, 128), 1)
    m = jnp.min(d2, axis=1, keepdims=True)
    cidx = jnp.min(jnp.where(d2 == m, lane, 999), axis=1, keepdims=True)
    oh = (lane == cidx).astype(jnp.float32)

    hi = jax.lax.Precision.HIGHEST
    ids = jnp.dot(oh, idsf_ref[...], precision=hi)
    cx = jnp.dot(oh, gx_ref[...], precision=hi)
    cy = jnp.dot(oh, gy_ref[...], precision=hi)
    cz = jnp.dot(oh, gz_ref[...], precision=hi)
    d = (xa - cx) ** 2 + (ya - cy) ** 2 + (za - cz) ** 2

    sub = lax.broadcasted_iota(jnp.int32, (BA, 1), 0)
    aid = pid * BA + sub
    valid = (ids >= 0.0) & (ids != aid.astype(jnp.float32))
    dm = jnp.where(valid, d, _BIG8)

    w1 = par_ref[0, 1]
    w3 = par_ref[0, 3]
    w4 = par_ref[0, 4]
    s_nl = 1.0 + par_ref[0, 5]

    th = _energy_terms(dm, w1, w3, w4)
    s_direct = jnp.sum(th, axis=1, keepdims=True)
    c8 = jnp.sum((dm < 8.0).astype(jnp.float32), axis=1, keepdims=True)
    e_ref[...] = s_direct

    @pl.when(jnp.max(c8) > 16.0)
    def _rank_fix():
        dcur = dm
        s = jnp.zeros((BA, 1), jnp.float32)
        taken = jnp.zeros((BA, 1), jnp.float32)
        for _ in range(M_NBRS):
            mv = jnp.min(dcur, axis=1, keepdims=True)
            cnt = jnp.sum((dcur == mv).astype(jnp.float32), axis=1, keepdims=True)
            take = jnp.clip(16.0 - taken, 0.0, cnt)
            s = s + take * _energy_terms(mv, w1, w3, w4)
            taken = taken + take
            dcur = jnp.where(dcur == mv, _BIG9, dcur)
        e_ref[...] = jnp.where(c8 > 16.0, s, s_direct)

    ok = (aid < N_ATOMS).astype(jnp.float32)
    blocksum = jnp.sum(e_ref[...] * ok)
    acc_ref[0, 0] = acc_ref[0, 0] + blocksum / s_nl
    out_ref[0, 0] = acc_ref[0, 0]


def _run_energy(xc_pad, idsf, gx, gy, gz, params):
    return pl.pallas_call(
        _energy_kernel,
        grid=(N_BLOCKS,),
        in_specs=[
            pl.BlockSpec((BA, 4), lambda i: (i, 0)),
            pl.BlockSpec((8, 128), lambda i: (0, 0)),
            pl.BlockSpec((128, 128), lambda i: (0, 0)),
            pl.BlockSpec((128, 128), lambda i: (0, 0)),
            pl.BlockSpec((128, 128), lambda i: (0, 0)),
            pl.BlockSpec((128, 128), lambda i: (0, 0)),
            pl.BlockSpec(memory_space=pltpu.SMEM),
        ],
        out_specs=pl.BlockSpec(memory_space=pltpu.SMEM),
        out_shape=jax.ShapeDtypeStruct((1, 1), jnp.float32),
        scratch_shapes=[
            pltpu.VMEM((BA, 1), jnp.float32),
            pltpu.SMEM((1, 1), jnp.float32),
        ],
    )(xc_pad, jnp.asarray(_CELLS_LANE), idsf, gx, gy, gz, params)


def kernel(X, Z, combo_w, nl_w):
    X = X.astype(jnp.float32)

    xt = jnp.concatenate(
        [X.T, jnp.full((3, N_SCAN - N_ATOMS), _FAR, jnp.float32)], axis=1)
    xt_pad = jnp.concatenate([xt, jnp.zeros((5, N_SCAN), jnp.float32)], axis=0)
    xc = jnp.concatenate([X, jnp.zeros((N_ATOMS, 1), jnp.float32)], axis=1)
    xc_pad = jnp.concatenate(
        [xc, jnp.zeros((N_EPAD - N_ATOMS, 4), jnp.float32)], axis=0)
    params = jnp.zeros((1, 8), jnp.float32)
    params = params.at[0, 0:5].set(combo_w).at[0, 5].set(nl_w[0])

    closest = _run_topk_cells(xt_pad)

    ain = closest[:N_CELLS][jnp.asarray(_NBR_CELLS)].reshape(N_CELLS, 128)
    ain = jnp.concatenate(
        [ain, jnp.full((3, 128), -1, jnp.int32)], axis=0)

    idsf = _run_dedup(ain)

    table128 = jnp.concatenate(
        [X, jnp.zeros((N_ATOMS, 125), jnp.float32)], axis=1)
    rows = _sc_gather(table128, jnp.maximum(ain, 0).reshape(_NGATHER))
    gx = rows[:, 0].reshape(128, 128)
    gy = rows[:, 1].reshape(128, 128)
    gz = rows[:, 2].reshape(128, 128)

    out = _run_energy(xc_pad, idsf, gx, gy, gz, params)
    return out[0, 0]


# speedup vs baseline: 66.8121x; 1.0116x over previous
import functools

import jax
import jax.numpy as jnp
import numpy as np
from jax import lax
from jax.experimental import pallas as pl
from jax.experimental.pallas import tpu as pltpu
from jax.experimental.pallas import tpu_sc as plsc

N_ATOMS = 50000
M_NBRS = 16
NBR_CUTOFF = 8.0
N_CELLS = 125

CHUNK = 2048
N_CHUNKS = 25
N_SCAN = N_CHUNKS * CHUNK

BA = 512
N_BLOCKS = 98
N_EPAD = N_BLOCKS * BA

_BIGF = np.float32(1e30)
_BIGI = np.int32(2**30)
_BIG8 = np.float32(1e9)
_BIG9 = np.float32(2e9)
_FAR = np.float32(1e5)

_r = np.arange(0.0, 40.0, 8.0, dtype=np.float32)
_mesh = np.meshgrid(_r, _r, _r)
_CELLS = np.transpose(np.stack(_mesh)).reshape(N_CELLS, 3).astype(np.float32)
_d_cc = ((_CELLS[:, None, :] - _CELLS[None, :, :]) ** 2).sum(-1)
_NBR_CELLS = np.argsort(_d_cc, axis=1, kind="stable")[:, :8].astype(np.int32)

_CELLS_LANE = np.zeros((8, 128), np.float32)
_CELLS_LANE[0:3, :N_CELLS] = _CELLS.T
_CELLS_LANE[0:3, N_CELLS:] = _FAR
_CELLS_SUB = np.zeros((128, 8), np.float32)
_CELLS_SUB[:N_CELLS, 0:3] = _CELLS
_CELLS_SUB[N_CELLS:, 0:3] = _FAR


def _topk_cells_kernel(xt_ref, csub_ref, out_ref, rv_ref, ri_ref):
    pid = pl.program_id(0)

    @pl.when(pid == 0)
    def _init():
        rv_ref[...] = jnp.full((128, 16), _BIGF, jnp.float32)
        ri_ref[...] = jnp.full((128, 16), _BIG8, jnp.float32)

    xs = xt_ref[0:1, :]
    ys = xt_ref[1:2, :]
    zs = xt_ref[2:3, :]
    cx = csub_ref[:, 0:1]
    cy = csub_ref[:, 1:2]
    cz = csub_ref[:, 2:3]
    d = (cx - xs) ** 2 + (cy - ys) ** 2 + (cz - zs) ** 2
    gidx = (pid * CHUNK).astype(jnp.float32) + lax.broadcasted_iota(
        jnp.int32, (128, CHUNK), 1).astype(jnp.float32)

    vals, idxs = [], []
    for _ in range(M_NBRS):
        m = jnp.min(d, axis=1, keepdims=True)
        mi = jnp.min(jnp.where(d == m, gidx, _BIG8), axis=1, keepdims=True)
        vals.append(m)
        idxs.append(mi)
        d = jnp.where(gidx == mi, _BIGF, d)
    av = jnp.concatenate([rv_ref[...]] + vals, axis=1)
    ai = jnp.concatenate([ri_ref[...]] + idxs, axis=1)

    nvals, nidxs = [], []
    for _ in range(M_NBRS):
        m = jnp.min(av, axis=1, keepdims=True)
        mi = jnp.min(jnp.where(av == m, ai, _BIG8), axis=1, keepdims=True)
        nvals.append(m)
        nidxs.append(mi)
        av = jnp.where((av == m) & (ai == mi), _BIGF, av)
    rv_ref[...] = jnp.concatenate(nvals, axis=1)
    ri_ref[...] = jnp.concatenate(nidxs, axis=1)
    out_ref[...] = ri_ref[...].astype(jnp.int32)


def _run_topk_cells(xt_pad):
    return pl.pallas_call(
        _topk_cells_kernel,
        grid=(N_CHUNKS,),
        in_specs=[
            pl.BlockSpec((8, CHUNK), lambda i: (0, i)),
            pl.BlockSpec((128, 8), lambda i: (0, 0)),
        ],
        out_specs=pl.BlockSpec((128, M_NBRS), lambda i: (0, 0)),
        out_shape=jax.ShapeDtypeStruct((128, M_NBRS), jnp.int32),
        scratch_shapes=[
            pltpu.VMEM((128, M_NBRS), jnp.float32),
            pltpu.VMEM((128, M_NBRS), jnp.float32),
        ],
    )(xt_pad, jnp.asarray(_CELLS_SUB))


def _dedup_kernel(ids_ref, idsf_ref):
    ids = ids_ref[...]
    lane = lax.broadcasted_iota(jnp.int32, (128, 128), 1)
    dup = jnp.zeros((128, 128), jnp.bool_)
    for j in range(1, 128):
        idj = ids[:, j:j + 1]
        dupj = jnp.any((ids == idj) & (lane < j), axis=1, keepdims=True)
        dup = dup | (dupj & (lane == j))
    clean = jnp.where(dup, -1, ids)
    idsf_ref[...] = clean.astype(jnp.float32)


def _run_dedup(ain):
    return pl.pallas_call(
        _dedup_kernel,
        out_shape=jax.ShapeDtypeStruct((128, 128), jnp.float32),
    )(ain)


_NGATHER = 128 * 128
_B_PER_W = _NGATHER // 32


def _sc_gather(table128, gidx_flat):
    mesh = plsc.VectorSubcoreMesh(
        core_axis_name="c", subcore_axis_name="s", num_cores=2, num_subcores=16
    )

    @functools.partial(
        pl.kernel,
        out_type=jax.ShapeDtypeStruct((_NGATHER, 128), jnp.float32),
        mesh=mesh,
        scratch_types=[
            pltpu.VMEM((_B_PER_W,), jnp.int32),
            pltpu.VMEM((_B_PER_W, 128), jnp.float32),
            pltpu.SemaphoreType.DMA,
        ],
    )
    def gat(table_hbm, idx_hbm, out_hbm, idx_v, rows_v, sem):
        wid = lax.axis_index("s") * 2 + lax.axis_index("c")
        base = wid * _B_PER_W
        pltpu.sync_copy(idx_hbm.at[pl.ds(base, _B_PER_W)], idx_v)
        pltpu.async_copy(table_hbm.at[idx_v], rows_v, sem).wait()
        pltpu.sync_copy(rows_v, out_hbm.at[pl.ds(base, _B_PER_W)])

    return gat(table128, gidx_flat)


def _energy_terms(d, w1, w3, w4):
    hyd = jnp.where(d < 0.5, jnp.ones_like(d),
                    jnp.where(d < 1.5, 1.5 - d, jnp.zeros_like(d)))
    g1 = jnp.exp(-((d / 0.5) ** 2))
    g2 = jnp.exp(-(((d - 3.0) / 2.0) ** 2))
    inter = w1 * hyd + w3 * g1 + w4 * g2
    return jnp.where(d < 8.0, inter, jnp.zeros_like(d))


def _energy_kernel(xc_ref, clane_ref, idsf_ref, gx_ref, gy_ref, gz_ref,
                   par_ref, out_ref, e_ref, acc_ref):
    pid = pl.program_id(0)

    @pl.when(pid == 0)
    def _init():
        acc_ref[0, 0] = jnp.float32(0.0)

    xa = xc_ref[:, 0:1]
    ya = xc_ref[:, 1:2]
    za = xc_ref[:, 2:3]
    cxr = clane_ref[0:1, :]
    cyr = clane_ref[1:2, :]
    czr = clane_ref[2:3, :]
    d2 = (xa - cxr) ** 2 + (ya - cyr) ** 2 + (za - czr) ** 2
    lane = lax.broadcasted_iota(jnp.int32, (BA, 128), 1).astype(jnp.float32)
    m = jnp.min(d2, axis=1, keepdims=True)
    cidx = jnp.min(jnp.where(d2 == m, lane, _BIG8), axis=1, keepdims=True)
    oh = (lane == cidx).astype(jnp.float32)

    hi = jax.lax.Precision.HIGHEST
    ids = jnp.dot(oh, idsf_ref[...], precision=hi)
    cx = jnp.dot(oh, gx_ref[...], precision=hi)
    cy = jnp.dot(oh, gy_ref[...], precision=hi)
    cz = jnp.dot(oh, gz_ref[...], precision=hi)
    d = (xa - cx) ** 2 + (ya - cy) ** 2 + (za - cz) ** 2

    sub = lax.broadcasted_iota(jnp.int32, (BA, 1), 0)
    aid = pid * BA + sub
    valid = (ids >= 0.0) & (ids != aid.astype(jnp.float32))

    w1 = par_ref[0, 1]
    w3 = par_ref[0, 3]
    w4 = par_ref[0, 4]
    s_nl = 1.0 + par_ref[0, 5]

    mask = valid & (d < 8.0)
    hyd = jnp.clip(1.5 - d, 0.0, 1.0)
    g1 = jnp.exp(-((d / 0.5) ** 2))
    g2 = jnp.exp(-(((d - 3.0) / 2.0) ** 2))
    inter = w1 * hyd + w3 * g1 + w4 * g2
    th = jnp.where(mask, inter, jnp.zeros_like(d))
    s_direct = jnp.sum(th, axis=1, keepdims=True)
    c8 = jnp.sum(mask.astype(jnp.float32), axis=1, keepdims=True)
    e_ref[...] = s_direct

    @pl.when(jnp.max(c8) > 16.0)
    def _rank_fix():
        dcur = jnp.where(valid, d, _BIG8)
        s = jnp.zeros((BA, 1), jnp.float32)
        taken = jnp.zeros((BA, 1), jnp.float32)
        for _ in range(M_NBRS):
            mv = jnp.min(dcur, axis=1, keepdims=True)
            cnt = jnp.sum((dcur == mv).astype(jnp.float32), axis=1, keepdims=True)
            take = jnp.clip(16.0 - taken, 0.0, cnt)
            s = s + take * _energy_terms(mv, w1, w3, w4)
            taken = taken + take
            dcur = jnp.where(dcur == mv, _BIG9, dcur)
        e_ref[...] = jnp.where(c8 > 16.0, s, s_direct)

    ok = (aid < N_ATOMS).astype(jnp.float32)
    blocksum = jnp.sum(e_ref[...] * ok)
    acc_ref[0, 0] = acc_ref[0, 0] + blocksum / s_nl
    out_ref[0, 0] = acc_ref[0, 0]


def _run_energy(xc_pad, idsf, gx, gy, gz, params):
    return pl.pallas_call(
        _energy_kernel,
        grid=(N_BLOCKS,),
        in_specs=[
            pl.BlockSpec((BA, 4), lambda i: (i, 0)),
            pl.BlockSpec((8, 128), lambda i: (0, 0)),
            pl.BlockSpec((128, 128), lambda i: (0, 0)),
            pl.BlockSpec((128, 128), lambda i: (0, 0)),
            pl.BlockSpec((128, 128), lambda i: (0, 0)),
            pl.BlockSpec((128, 128), lambda i: (0, 0)),
            pl.BlockSpec(memory_space=pltpu.SMEM),
        ],
        out_specs=pl.BlockSpec(memory_space=pltpu.SMEM),
        out_shape=jax.ShapeDtypeStruct((1, 1), jnp.float32),
        scratch_shapes=[
            pltpu.VMEM((BA, 1), jnp.float32),
            pltpu.SMEM((1, 1), jnp.float32),
        ],
    )(xc_pad, jnp.asarray(_CELLS_LANE), idsf, gx, gy, gz, params)


def kernel(X, Z, combo_w, nl_w):
    X = X.astype(jnp.float32)

    xt = jnp.concatenate(
        [X.T, jnp.full((3, N_SCAN - N_ATOMS), _FAR, jnp.float32)], axis=1)
    xt_pad = jnp.concatenate([xt, jnp.zeros((5, N_SCAN), jnp.float32)], axis=0)
    xc = jnp.concatenate([X, jnp.zeros((N_ATOMS, 1), jnp.float32)], axis=1)
    xc_pad = jnp.concatenate(
        [xc, jnp.zeros((N_EPAD - N_ATOMS, 4), jnp.float32)], axis=0)
    params = jnp.zeros((1, 8), jnp.float32)
    params = params.at[0, 0:5].set(combo_w).at[0, 5].set(nl_w[0])

    closest = _run_topk_cells(xt_pad)

    ain = closest[:N_CELLS][jnp.asarray(_NBR_CELLS)].reshape(N_CELLS, 128)
    ain = jnp.concatenate(
        [ain, jnp.full((3, 128), -1, jnp.int32)], axis=0)

    idsf = _run_dedup(ain)

    table128 = jnp.concatenate(
        [X, jnp.zeros((N_ATOMS, 125), jnp.float32)], axis=1)
    rows = _sc_gather(table128, jnp.maximum(ain, 0).reshape(_NGATHER))
    gx = rows[:, 0].reshape(128, 128)
    gy = rows[:, 1].reshape(128, 128)
    gz = rows[:, 2].reshape(128, 128)

    out = _run_energy(xc_pad, idsf, gx, gy, gz, params)
    return out[0, 0]
